# baseline (speedup 1.0000x reference)
import functools

import jax
import jax.numpy as jnp
from jax import lax
from jax.experimental import pallas as pl
from jax.experimental.pallas import tpu as pltpu
from jax.experimental.pallas import tpu_sc as plsc

_L = 16
_NC = 2
_NS = 16


@functools.lru_cache(maxsize=None)
def _make_sc_argmax(B, S, V):
    NW = _NC * _NS
    RPW = B // NW
    CH = V // _L
    mesh = plsc.VectorSubcoreMesh(core_axis_name="c", subcore_axis_name="s")

    @functools.partial(
        pl.kernel,
        out_type=(
            jax.ShapeDtypeStruct((NW, _L), jnp.int32),
            jax.ShapeDtypeStruct((B, V), jnp.float32),
        ),
        mesh=mesh,
        scratch_types=[
            pltpu.VMEM((RPW, V), jnp.float32),
            pltpu.VMEM((_L,), jnp.int32),
            pltpu.SemaphoreType.DMA,
            pltpu.SemaphoreType.DMA,
        ],
    )
    def k(probas_hbm, cand_hbm, dist_hbm, rows_v, cand_v, in_sem, out_sem):
        wid = lax.axis_index("s") * _NC + lax.axis_index("c")
        base = wid * RPW
        loads = [
            pltpu.async_copy(probas_hbm.at[base + r, S - 1], rows_v.at[r], in_sem)
            for r in range(RPW)
        ]
        for ld in loads:
            ld.wait()
        st = pltpu.async_copy(rows_v, dist_hbm.at[pl.ds(base, RPW)], out_sem)

        lane = lax.iota(jnp.int32, _L)
        cand_vec = jnp.zeros((_L,), jnp.int32)
        for r in range(RPW):
            def body(i, carry, r=r):
                maxv, maxi, idx = carry
                v = rows_v[r, pl.ds(i * _L, _L)]
                upd = v > maxv
                return (
                    jnp.where(upd, v, maxv),
                    jnp.where(upd, idx, maxi),
                    idx + _L,
                )
            maxv, maxi, _ = lax.fori_loop(
                0, CH, body,
                (jnp.full((_L,), -jnp.inf, jnp.float32),
                 jnp.zeros((_L,), jnp.int32),
                 lane),
                unroll=4,
            )
            for s in (1, 2, 4, 8):
                perm = jnp.bitwise_xor(lane, s)
                ov = maxv.at[perm].get(mode="promise_in_bounds")
                oi = maxi.at[perm].get(mode="promise_in_bounds")
                upd = (ov > maxv) | ((ov == maxv) & (oi < maxi))
                maxv = jnp.where(upd, ov, maxv)
                maxi = jnp.where(upd, oi, maxi)
            cand_vec = jnp.where(lane == r, maxi, cand_vec)
        cand_v[...] = cand_vec
        pltpu.sync_copy(cand_v, cand_hbm.at[wid])
        st.wait()

    return k, RPW


def kernel(probas, greedy):
    B, S, V = probas.shape
    k, rpw = _make_sc_argmax(B, S, V)
    cand_pad, dist = k(probas)
    candidate = cand_pad[:, :rpw].reshape(B, 1)
    return (candidate, dist)


# speedup vs baseline: 1.0285x; 1.0285x over previous
import functools

import jax
import jax.numpy as jnp
from jax import lax
from jax.experimental import pallas as pl
from jax.experimental.pallas import tpu as pltpu
from jax.experimental.pallas import tpu_sc as plsc

_L = 16
_NC = 2
_NS = 16


@functools.lru_cache(maxsize=None)
def _make_sc_argmax(B, S, V):
    NW = _NC * _NS
    RPW = B // NW
    BPC = B // _NC
    CH = V // _L
    mesh = plsc.VectorSubcoreMesh(core_axis_name="c", subcore_axis_name="s")

    @functools.partial(
        pl.kernel,
        out_type=(
            jax.ShapeDtypeStruct((B,), jnp.int32),
            jax.ShapeDtypeStruct((B, V), jnp.float32),
        ),
        mesh=mesh,
        scratch_types=[
            pltpu.VMEM((RPW, V), jnp.float32),
            pltpu.VMEM((_L,), jnp.int32),
            pltpu.VMEM_SHARED((_NS * _L,), jnp.int32),
            pltpu.VMEM((RPW, _L), jnp.int32),
            pltpu.VMEM((_L,), jnp.int32),
            pltpu.SemaphoreType.DMA((RPW,)),
            pltpu.SemaphoreType.DMA,
        ],
    )
    def k(probas_hbm, cand_hbm, dist_hbm, rows_v, cand_v, shared_c, quad_v,
          out_c, in_sems, out_sem):
        cid = lax.axis_index("c")
        sid = lax.axis_index("s")
        wid = cid * _NS + sid
        base = wid * RPW
        loads = [
            pltpu.async_copy(probas_hbm.at[base + r, S - 1], rows_v.at[r],
                             in_sems.at[r])
            for r in range(RPW)
        ]
        lane = lax.iota(jnp.int32, _L)
        cand_vec = jnp.zeros((_L,), jnp.int32)
        stores = []
        for r in range(RPW):
            loads[r].wait()
            stores.append(
                pltpu.async_copy(rows_v.at[r], dist_hbm.at[base + r], out_sem))

            def body(i, carry, r=r):
                maxv, maxi, idx = carry
                v = rows_v[r, pl.ds(i * _L, _L)]
                upd = v > maxv
                return (
                    jnp.where(upd, v, maxv),
                    jnp.where(upd, idx, maxi),
                    idx + _L,
                )
            maxv, maxi, _ = lax.fori_loop(
                0, CH, body,
                (jnp.full((_L,), -jnp.inf, jnp.float32),
                 jnp.zeros((_L,), jnp.int32),
                 lane),
                unroll=4,
            )
            for s in (1, 2, 4, 8):
                perm = jnp.bitwise_xor(lane, s)
                ov = maxv.at[perm].get(mode="promise_in_bounds")
                oi = maxi.at[perm].get(mode="promise_in_bounds")
                upd = (ov > maxv) | ((ov == maxv) & (oi < maxi))
                maxv = jnp.where(upd, ov, maxv)
                maxi = jnp.where(upd, oi, maxi)
            cand_vec = jnp.where(lane == r, maxi, cand_vec)
        cand_v[...] = cand_vec
        pltpu.sync_copy(cand_v, shared_c.at[pl.ds(sid * _L, _L)])
        plsc.subcore_barrier()

        t = sid & (_L // RPW - 1)
        for j in range(RPW):
            pltpu.sync_copy(shared_c.at[pl.ds((t * RPW + j) * _L, _L)],
                            quad_v.at[j])
        parts = [quad_v[j] for j in range(RPW)]
        merged = parts[0]
        for j in range(1, RPW):
            shuf = parts[j].at[(lane - j * RPW) & (_L - 1)].get(
                mode="promise_in_bounds")
            merged = jnp.where(lane < j * RPW, merged, shuf)
        out_c[...] = merged

        @pl.when(sid < _L // RPW)
        def _flush():
            pltpu.sync_copy(out_c, cand_hbm.at[pl.ds(cid * BPC + t * _L, _L)])

        for st in stores:
            st.wait()

    return k


def kernel(probas, greedy):
    B, S, V = probas.shape
    cand, dist = _make_sc_argmax(B, S, V)(probas)
    return (cand.reshape(B, 1), dist)
